# baseline (speedup 1.0000x reference)
import jax
import jax.numpy as jnp
from jax.experimental import pallas as pl
from jax.experimental.pallas import tpu as pltpu

N_DOCS = 5
NCAP = 3
Q = 1024
D = 128
K = 100000
BK = 4096
NK = (K + BK - 1) // BK
KPAD = NK * BK
CHUNK = 128
NCH = BK // CHUNK

NEG_INF = float("-inf")
IMAX = jnp.iinfo(jnp.int32).max


def _dot(q, k):
    return jax.lax.dot_general(
        q, k, dimension_numbers=(((1,), (1,)), ((), ())),
        preferred_element_type=jnp.float32)


NCH_TAIL = (K - (NK - 1) * BK + CHUNK - 1) // CHUNK
COL_IOTA = None


def _insert3(w, wid, tv_ref, ti_ref, v4_ref):
    for t in range(NCAP):
        tv = tv_ref[t]
        ti = ti_ref[t]
        gt = w > tv
        tv_ref[t] = jnp.maximum(tv, w)
        ti_ref[t] = jnp.where(gt, wid, ti)
        if t < NCAP - 1:
            w, wid = jnp.minimum(tv, w), jnp.where(gt, ti, wid)
        else:
            w = jnp.minimum(tv, w)
    v4_ref[...] = jnp.maximum(v4_ref[...], w)


def _sweep3_body(q_ref, k_ref, tv_out, ti_out, v4_out, tv_ref, ti_ref, v4_ref):
    kb = pl.program_id(0)

    @pl.when(kb == 0)
    def _init():
        tv_ref[...] = jnp.full(tv_ref.shape, NEG_INF, jnp.float32)
        ti_ref[...] = jnp.zeros(ti_ref.shape, jnp.int32)
        v4_ref[...] = jnp.full(v4_ref.shape, NEG_INF, jnp.float32)

    s = _dot(q_ref[...], k_ref[...])

    @pl.when(kb < NK - 1)
    def _full_block():
        for r in range(NCH):
            _insert3(s[:, r * CHUNK:(r + 1) * CHUNK], kb * NCH + r,
                     tv_ref, ti_ref, v4_ref)

    @pl.when(kb == NK - 1)
    def _tail_block():
        col = jax.lax.broadcasted_iota(jnp.int32, (Q, CHUNK), 1)
        for r in range(NCH_TAIL):
            limit = K - (NK - 1) * BK - r * CHUNK
            w = s[:, r * CHUNK:(r + 1) * CHUNK]
            if limit < CHUNK:
                w = jnp.where(col < limit, w, NEG_INF)
            _insert3(w, (NK - 1) * NCH + r, tv_ref, ti_ref, v4_ref)
        tv_out[...] = tv_ref[...]
        ti_out[...] = ti_ref[...]
        v4_out[...] = v4_ref[...]


def _merge3_body(tv_ref, ti_ref, v4_ref, out_v_ref, out_i_ref, flag_ref):
    cv = jnp.concatenate([tv_ref[t] for t in range(NCAP)], axis=1)
    cc = jnp.concatenate([ti_ref[t] for t in range(NCAP)], axis=1)
    lane = jax.lax.rem(
        jax.lax.broadcasted_iota(jnp.int32, (Q, NCAP * CHUNK), 1), CHUNK)
    ci = cc * CHUNK + lane
    cv = jnp.where(ci >= K, NEG_INF, cv)
    x5 = None
    for t in range(N_DOCS):
        m = jnp.max(cv, axis=1, keepdims=True)
        hit = cv == m
        sel = jnp.min(jnp.where(hit, ci, IMAX), axis=1, keepdims=True)
        out_v_ref[:, pl.ds(t, 1)] = m
        out_i_ref[:, pl.ds(t, 1)] = sel
        cv = jnp.where(hit & (ci == sel), NEG_INF, cv)
        x5 = m
    mv4 = jnp.max(v4_ref[...], axis=1, keepdims=True)
    n_suspect = jnp.sum((mv4 >= x5).astype(jnp.int32))
    flag_ref[...] = jnp.broadcast_to(n_suspect, flag_ref.shape)


def _insert5(w, wid, tv_ref, ti_ref):
    for t in range(N_DOCS):
        tv = tv_ref[t]
        ti = ti_ref[t]
        gt = w > tv
        tv_ref[t] = jnp.maximum(tv, w)
        ti_ref[t] = jnp.where(gt, wid, ti)
        if t < N_DOCS - 1:
            w, wid = jnp.minimum(tv, w), jnp.where(gt, ti, wid)


def _sweep5_body(q_ref, k_ref, tv_out, ti_out, tv_ref, ti_ref):
    kb = pl.program_id(0)

    @pl.when(kb == 0)
    def _init():
        tv_ref[...] = jnp.full(tv_ref.shape, NEG_INF, jnp.float32)
        ti_ref[...] = jnp.zeros(ti_ref.shape, jnp.int32)

    s = _dot(q_ref[...], k_ref[...])

    @pl.when(kb < NK - 1)
    def _full_block():
        for r in range(NCH):
            _insert5(s[:, r * CHUNK:(r + 1) * CHUNK], kb * NCH + r,
                     tv_ref, ti_ref)

    @pl.when(kb == NK - 1)
    def _tail_block():
        col = jax.lax.broadcasted_iota(jnp.int32, (Q, CHUNK), 1)
        for r in range(NCH_TAIL):
            limit = K - (NK - 1) * BK - r * CHUNK
            w = s[:, r * CHUNK:(r + 1) * CHUNK]
            if limit < CHUNK:
                w = jnp.where(col < limit, w, NEG_INF)
            _insert5(w, (NK - 1) * NCH + r, tv_ref, ti_ref)
        tv_out[...] = tv_ref[...]
        ti_out[...] = ti_ref[...]


def _merge5_body(tv_ref, ti_ref, out_v_ref, out_i_ref):
    cv = jnp.concatenate([tv_ref[t] for t in range(N_DOCS)], axis=1)
    cc = jnp.concatenate([ti_ref[t] for t in range(N_DOCS)], axis=1)
    lane = jax.lax.rem(
        jax.lax.broadcasted_iota(jnp.int32, (Q, N_DOCS * CHUNK), 1), CHUNK)
    ci = cc * CHUNK + lane
    cv = jnp.where(ci >= K, NEG_INF, cv)
    for t in range(N_DOCS):
        m = jnp.max(cv, axis=1, keepdims=True)
        hit = cv == m
        sel = jnp.min(jnp.where(hit, ci, IMAX), axis=1, keepdims=True)
        out_v_ref[:, pl.ds(t, 1)] = m
        out_i_ref[:, pl.ds(t, 1)] = sel
        cv = jnp.where(hit & (ci == sel), NEG_INF, cv)


def _run_sweep(body, depth, queries, keys_p):
    return pl.pallas_call(
        body,
        grid=(NK,),
        in_specs=[
            pl.BlockSpec((Q, D), lambda k: (0, 0)),
            pl.BlockSpec((BK, D), lambda k: (k, 0)),
        ],
        out_specs=[
            pl.BlockSpec((depth, Q, CHUNK), lambda k: (0, 0, 0)),
            pl.BlockSpec((depth, Q, CHUNK), lambda k: (0, 0, 0)),
        ] + ([pl.BlockSpec((Q, CHUNK), lambda k: (0, 0))]
             if depth == NCAP else []),
        out_shape=[
            jax.ShapeDtypeStruct((depth, Q, CHUNK), jnp.float32),
            jax.ShapeDtypeStruct((depth, Q, CHUNK), jnp.int32),
        ] + ([jax.ShapeDtypeStruct((Q, CHUNK), jnp.float32)]
             if depth == NCAP else []),
        scratch_shapes=[
            pltpu.VMEM((depth, Q, CHUNK), jnp.float32),
            pltpu.VMEM((depth, Q, CHUNK), jnp.int32),
        ] + ([pltpu.VMEM((Q, CHUNK), jnp.float32)] if depth == NCAP else []),
        compiler_params=pltpu.CompilerParams(
            dimension_semantics=("arbitrary",),
        ),
    )(queries, keys_p)


def kernel(queries, keys):
    tv, ti, v4 = _run_sweep(_sweep3_body, NCAP, queries, keys)
    out_v, out_i, flag = pl.pallas_call(
        _merge3_body,
        out_shape=[
            jax.ShapeDtypeStruct((Q, N_DOCS), jnp.float32),
            jax.ShapeDtypeStruct((Q, N_DOCS), jnp.int32),
            jax.ShapeDtypeStruct((8, 128), jnp.int32),
        ],
    )(tv, ti, v4)

    def _slow_path(_):
        tv5, ti5 = _run_sweep(_sweep5_body, N_DOCS, queries, keys)
        return pl.pallas_call(
            _merge5_body,
            out_shape=[
                jax.ShapeDtypeStruct((Q, N_DOCS), jnp.float32),
                jax.ShapeDtypeStruct((Q, N_DOCS), jnp.int32),
            ],
        )(tv5, ti5)

    return jax.lax.cond(
        flag[0, 0] > 0, _slow_path, lambda _: (out_v, out_i), None)


# speedup vs baseline: 3.4168x; 3.4168x over previous
import jax
import jax.numpy as jnp
from jax.experimental import pallas as pl
from jax.experimental.pallas import tpu as pltpu

N_DOCS = 5
NCAP = 3
Q = 1024
D = 128
K = 100000
BK = 4096
NK_MAIN = K // BK
MAIN = NK_MAIN * BK
TAIL = K - MAIN
BT = 2048
CHUNK = 128
NCH = BK // CHUNK
NCH_TAIL = (TAIL + CHUNK - 1) // CHUNK

NEG_INF = float("-inf")
IMAX = jnp.iinfo(jnp.int32).max


def _dot(q, k):
    return jax.lax.dot_general(
        q, k, dimension_numbers=(((1,), (1,)), ((), ())),
        preferred_element_type=jnp.float32)


def _sweep3_body(q_ref, k_ref, tv_out, ti_out, v4_out, tv_ref, ti_ref, v4_ref):
    kb = pl.program_id(0)

    @pl.when(kb == 0)
    def _init():
        tv_ref[...] = jnp.full(tv_ref.shape, NEG_INF, jnp.float32)
        ti_ref[...] = jnp.zeros(ti_ref.shape, jnp.int32)
        v4_ref[...] = jnp.full(v4_ref.shape, NEG_INF, jnp.float32)

    s = _dot(q_ref[...], k_ref[...])

    for r in range(NCH):
        w = s[:, r * CHUNK:(r + 1) * CHUNK]
        wid = kb * NCH + r
        for t in range(NCAP):
            tv = tv_ref[t]
            ti = ti_ref[t]
            gt = w > tv
            tv_ref[t] = jnp.maximum(tv, w)
            ti_ref[t] = jnp.where(gt, wid, ti)
            if t < NCAP - 1:
                w, wid = jnp.minimum(tv, w), jnp.where(gt, ti, wid)
            else:
                w = jnp.minimum(tv, w)
        v4_ref[...] = jnp.maximum(v4_ref[...], w)

    @pl.when(kb == NK_MAIN - 1)
    def _flush():
        tv_out[...] = tv_ref[...]
        ti_out[...] = ti_ref[...]
        v4_out[...] = v4_ref[...]


def _sweep5_body(q_ref, k_ref, tv_out, ti_out, tv_ref, ti_ref):
    kb = pl.program_id(0)

    @pl.when(kb == 0)
    def _init():
        tv_ref[...] = jnp.full(tv_ref.shape, NEG_INF, jnp.float32)
        ti_ref[...] = jnp.zeros(ti_ref.shape, jnp.int32)

    s = _dot(q_ref[...], k_ref[...])

    for r in range(NCH):
        w = s[:, r * CHUNK:(r + 1) * CHUNK]
        wid = kb * NCH + r
        for t in range(N_DOCS):
            tv = tv_ref[t]
            ti = ti_ref[t]
            gt = w > tv
            tv_ref[t] = jnp.maximum(tv, w)
            ti_ref[t] = jnp.where(gt, wid, ti)
            if t < N_DOCS - 1:
                w, wid = jnp.minimum(tv, w), jnp.where(gt, ti, wid)

    @pl.when(kb == NK_MAIN - 1)
    def _flush():
        tv_out[...] = tv_ref[...]
        ti_out[...] = ti_ref[...]


def _tail_scores(q_ref, kt_ref):
    s = _dot(q_ref[...], kt_ref[...])
    col = jax.lax.broadcasted_iota(jnp.int32, (Q, CHUNK), 1)
    out = []
    for r in range(NCH_TAIL):
        limit = TAIL - r * CHUNK
        w = s[:, r * CHUNK:(r + 1) * CHUNK]
        if limit < CHUNK:
            w = jnp.where(col < limit, w, NEG_INF)
        out.append((w, MAIN // CHUNK + r))
    return out


def _merge3_body(tv_ref, ti_ref, v4_ref, q_ref, kt_ref,
                 out_v_ref, out_i_ref, flag_ref):
    tvs = [tv_ref[t] for t in range(NCAP)]
    tis = [ti_ref[t] for t in range(NCAP)]
    v4 = v4_ref[...]
    for w, wid in _tail_scores(q_ref, kt_ref):
        for t in range(NCAP):
            gt = w > tvs[t]
            tvs[t], w = jnp.maximum(tvs[t], w), jnp.minimum(tvs[t], w)
            tis[t], wid = (jnp.where(gt, wid, tis[t]),
                           jnp.where(gt, tis[t], wid))
        v4 = jnp.maximum(v4, w)

    cv = jnp.concatenate(tvs, axis=1)
    cc = jnp.concatenate(tis, axis=1)
    lane = jax.lax.rem(
        jax.lax.broadcasted_iota(jnp.int32, (Q, NCAP * CHUNK), 1), CHUNK)
    ci = cc * CHUNK + lane
    x5 = None
    for t in range(N_DOCS):
        m = jnp.max(cv, axis=1, keepdims=True)
        hit = cv == m
        sel = jnp.min(jnp.where(hit, ci, IMAX), axis=1, keepdims=True)
        out_v_ref[:, pl.ds(t, 1)] = m
        out_i_ref[:, pl.ds(t, 1)] = sel
        cv = jnp.where(hit & (ci == sel), NEG_INF, cv)
        x5 = m
    mv4 = jnp.max(v4, axis=1, keepdims=True)
    n_suspect = jnp.sum((mv4 >= x5).astype(jnp.int32))
    flag_ref[...] = jnp.broadcast_to(n_suspect, flag_ref.shape)


def _merge5_body(tv_ref, ti_ref, q_ref, kt_ref, out_v_ref, out_i_ref):
    tvs = [tv_ref[t] for t in range(N_DOCS)]
    tis = [ti_ref[t] for t in range(N_DOCS)]
    for w, wid in _tail_scores(q_ref, kt_ref):
        for t in range(N_DOCS):
            gt = w > tvs[t]
            tvs[t], w = jnp.maximum(tvs[t], w), jnp.minimum(tvs[t], w)
            tis[t], wid = (jnp.where(gt, wid, tis[t]),
                           jnp.where(gt, tis[t], wid))

    cv = jnp.concatenate(tvs, axis=1)
    cc = jnp.concatenate(tis, axis=1)
    lane = jax.lax.rem(
        jax.lax.broadcasted_iota(jnp.int32, (Q, N_DOCS * CHUNK), 1), CHUNK)
    ci = cc * CHUNK + lane
    for t in range(N_DOCS):
        m = jnp.max(cv, axis=1, keepdims=True)
        hit = cv == m
        sel = jnp.min(jnp.where(hit, ci, IMAX), axis=1, keepdims=True)
        out_v_ref[:, pl.ds(t, 1)] = m
        out_i_ref[:, pl.ds(t, 1)] = sel
        cv = jnp.where(hit & (ci == sel), NEG_INF, cv)


def _run_sweep(body, depth, queries, keys):
    return pl.pallas_call(
        body,
        grid=(NK_MAIN,),
        in_specs=[
            pl.BlockSpec((Q, D), lambda k: (0, 0)),
            pl.BlockSpec((BK, D), lambda k: (k, 0)),
        ],
        out_specs=[
            pl.BlockSpec((depth, Q, CHUNK), lambda k: (0, 0, 0)),
            pl.BlockSpec((depth, Q, CHUNK), lambda k: (0, 0, 0)),
        ] + ([pl.BlockSpec((Q, CHUNK), lambda k: (0, 0))]
             if depth == NCAP else []),
        out_shape=[
            jax.ShapeDtypeStruct((depth, Q, CHUNK), jnp.float32),
            jax.ShapeDtypeStruct((depth, Q, CHUNK), jnp.int32),
        ] + ([jax.ShapeDtypeStruct((Q, CHUNK), jnp.float32)]
             if depth == NCAP else []),
        scratch_shapes=[
            pltpu.VMEM((depth, Q, CHUNK), jnp.float32),
            pltpu.VMEM((depth, Q, CHUNK), jnp.int32),
        ] + ([pltpu.VMEM((Q, CHUNK), jnp.float32)] if depth == NCAP else []),
        compiler_params=pltpu.CompilerParams(
            dimension_semantics=("arbitrary",),
        ),
    )(queries, keys)


def kernel(queries, keys):
    keys_tail = jnp.concatenate(
        [jax.lax.slice(keys, (MAIN, 0), (K, D)),
         jnp.zeros((BT - TAIL, D), jnp.float32)], axis=0)

    tv, ti, v4 = _run_sweep(_sweep3_body, NCAP, queries, keys)
    out_v, out_i, flag = pl.pallas_call(
        _merge3_body,
        out_shape=[
            jax.ShapeDtypeStruct((Q, N_DOCS), jnp.float32),
            jax.ShapeDtypeStruct((Q, N_DOCS), jnp.int32),
            jax.ShapeDtypeStruct((8, 128), jnp.int32),
        ],
    )(tv, ti, v4, queries, keys_tail)

    def _slow_path(_):
        tv5, ti5 = _run_sweep(_sweep5_body, N_DOCS, queries, keys)
        return pl.pallas_call(
            _merge5_body,
            out_shape=[
                jax.ShapeDtypeStruct((Q, N_DOCS), jnp.float32),
                jax.ShapeDtypeStruct((Q, N_DOCS), jnp.int32),
            ],
        )(tv5, ti5, queries, keys_tail)

    return jax.lax.cond(
        flag[0, 0] > 0, _slow_path, lambda _: (out_v, out_i), None)


# speedup vs baseline: 3.6862x; 1.0788x over previous
import jax
import jax.numpy as jnp
from jax.experimental import pallas as pl
from jax.experimental.pallas import tpu as pltpu

N_DOCS = 5
NCAP = 2
QR = 16
Q = 1024
D = 128
K = 100000
BK = 4096
NK_MAIN = K // BK
MAIN = NK_MAIN * BK
TAIL = K - MAIN
BT = 2048
CHUNK = 128
NCH = BK // CHUNK
NCH_TAIL = (TAIL + CHUNK - 1) // CHUNK

NEG_INF = float("-inf")
IMAX = jnp.iinfo(jnp.int32).max


def _dot(q, k):
    return jax.lax.dot_general(
        q, k, dimension_numbers=(((1,), (1,)), ((), ())),
        preferred_element_type=jnp.float32)


def _sweep2_body(q_ref, k_ref, tv_out, ti_out, v3_out, tv_ref, ti_ref, v3_ref):
    kb = pl.program_id(0)

    @pl.when(kb == 0)
    def _init():
        tv_ref[...] = jnp.full(tv_ref.shape, NEG_INF, jnp.float32)
        ti_ref[...] = jnp.zeros(ti_ref.shape, jnp.int32)
        v3_ref[...] = jnp.full(v3_ref.shape, NEG_INF, jnp.float32)

    s = _dot(q_ref[...], k_ref[...])

    for r in range(NCH):
        w = s[:, r * CHUNK:(r + 1) * CHUNK]
        wid = kb * NCH + r
        for t in range(NCAP):
            tv = tv_ref[t]
            ti = ti_ref[t]
            gt = w > tv
            tv_ref[t] = jnp.maximum(tv, w)
            ti_ref[t] = jnp.where(gt, wid, ti)
            if t < NCAP - 1:
                w, wid = jnp.minimum(tv, w), jnp.where(gt, ti, wid)
            else:
                w = jnp.minimum(tv, w)
        v3_ref[...] = jnp.maximum(v3_ref[...], w)

    @pl.when(kb == NK_MAIN - 1)
    def _flush():
        tv_out[...] = tv_ref[...]
        ti_out[...] = ti_ref[...]
        v3_out[...] = v3_ref[...]


def _sweep5_body(q_ref, k_ref, tv_out, ti_out, tv_ref, ti_ref):
    kb = pl.program_id(0)

    @pl.when(kb == 0)
    def _init():
        tv_ref[...] = jnp.full(tv_ref.shape, NEG_INF, jnp.float32)
        ti_ref[...] = jnp.zeros(ti_ref.shape, jnp.int32)

    s = _dot(q_ref[...], k_ref[...])

    for r in range(NCH):
        w = s[:, r * CHUNK:(r + 1) * CHUNK]
        wid = kb * NCH + r
        for t in range(N_DOCS):
            tv = tv_ref[t]
            ti = ti_ref[t]
            gt = w > tv
            tv_ref[t] = jnp.maximum(tv, w)
            ti_ref[t] = jnp.where(gt, wid, ti)
            if t < N_DOCS - 1:
                w, wid = jnp.minimum(tv, w), jnp.where(gt, ti, wid)

    @pl.when(kb == NK_MAIN - 1)
    def _flush():
        tv_out[...] = tv_ref[...]
        ti_out[...] = ti_ref[...]


def _tail_scores(q_ref, kt_ref):
    nq = q_ref.shape[0]
    s = _dot(q_ref[...], kt_ref[...])
    col = jax.lax.broadcasted_iota(jnp.int32, (nq, CHUNK), 1)
    out = []
    for r in range(NCH_TAIL):
        limit = TAIL - r * CHUNK
        w = s[:, r * CHUNK:(r + 1) * CHUNK]
        if limit < CHUNK:
            w = jnp.where(col < limit, w, NEG_INF)
        out.append((w, MAIN // CHUNK + r))
    return out


def _select_top5(tvs, tis, out_v_ref, out_i_ref):
    nq = tvs[0].shape[0]
    depth = len(tvs)
    cv = jnp.concatenate(tvs, axis=1)
    cc = jnp.concatenate(tis, axis=1)
    lane = jax.lax.rem(
        jax.lax.broadcasted_iota(jnp.int32, (nq, depth * CHUNK), 1), CHUNK)
    ci = cc * CHUNK + lane
    x5 = None
    for t in range(N_DOCS):
        m = jnp.max(cv, axis=1, keepdims=True)
        hit = cv == m
        sel = jnp.min(jnp.where(hit, ci, IMAX), axis=1, keepdims=True)
        out_v_ref[:, pl.ds(t, 1)] = m
        out_i_ref[:, pl.ds(t, 1)] = sel
        cv = jnp.where(hit & (ci == sel), NEG_INF, cv)
        x5 = m
    return x5


def _merge2_body(tv_ref, ti_ref, v3_ref, q_ref, kt_ref,
                 out_v_ref, out_i_ref, sus_ref):
    tvs = [tv_ref[t] for t in range(NCAP)]
    tis = [ti_ref[t] for t in range(NCAP)]
    v3 = v3_ref[...]
    for w, wid in _tail_scores(q_ref, kt_ref):
        for t in range(NCAP):
            gt = w > tvs[t]
            tvs[t], w = jnp.maximum(tvs[t], w), jnp.minimum(tvs[t], w)
            tis[t], wid = (jnp.where(gt, wid, tis[t]),
                           jnp.where(gt, tis[t], wid))
        v3 = jnp.maximum(v3, w)

    x5 = _select_top5(tvs, tis, out_v_ref, out_i_ref)
    mv3 = jnp.max(v3, axis=1, keepdims=True)
    sus_ref[...] = (mv3 >= x5).astype(jnp.int32)


def _merge5_body(tv_ref, ti_ref, q_ref, kt_ref, out_v_ref, out_i_ref):
    depth = tv_ref.shape[0]
    tvs = [tv_ref[t] for t in range(depth)]
    tis = [ti_ref[t] for t in range(depth)]
    for w, wid in _tail_scores(q_ref, kt_ref):
        for t in range(depth):
            gt = w > tvs[t]
            tvs[t], w = jnp.maximum(tvs[t], w), jnp.minimum(tvs[t], w)
            tis[t], wid = (jnp.where(gt, wid, tis[t]),
                           jnp.where(gt, tis[t], wid))
    _select_top5(tvs, tis, out_v_ref, out_i_ref)


def _run_sweep(body, depth, queries, keys, with_aux):
    nq = queries.shape[0]
    aux_out = ([pl.BlockSpec((nq, CHUNK), lambda k: (0, 0))]
               if with_aux else [])
    aux_shape = ([jax.ShapeDtypeStruct((nq, CHUNK), jnp.float32)]
                 if with_aux else [])
    aux_scratch = ([pltpu.VMEM((nq, CHUNK), jnp.float32)] if with_aux else [])
    return pl.pallas_call(
        body,
        grid=(NK_MAIN,),
        in_specs=[
            pl.BlockSpec((nq, D), lambda k: (0, 0)),
            pl.BlockSpec((BK, D), lambda k: (k, 0)),
        ],
        out_specs=[
            pl.BlockSpec((depth, nq, CHUNK), lambda k: (0, 0, 0)),
            pl.BlockSpec((depth, nq, CHUNK), lambda k: (0, 0, 0)),
        ] + aux_out,
        out_shape=[
            jax.ShapeDtypeStruct((depth, nq, CHUNK), jnp.float32),
            jax.ShapeDtypeStruct((depth, nq, CHUNK), jnp.int32),
        ] + aux_shape,
        scratch_shapes=[
            pltpu.VMEM((depth, nq, CHUNK), jnp.float32),
            pltpu.VMEM((depth, nq, CHUNK), jnp.int32),
        ] + aux_scratch,
        compiler_params=pltpu.CompilerParams(
            dimension_semantics=("arbitrary",),
        ),
    )(queries, keys)


def _depth5_topk(queries_sub, keys, keys_tail):
    tv5, ti5 = _run_sweep(_sweep5_body, N_DOCS, queries_sub, keys, False)
    nq = queries_sub.shape[0]
    return pl.pallas_call(
        _merge5_body,
        out_shape=[
            jax.ShapeDtypeStruct((nq, N_DOCS), jnp.float32),
            jax.ShapeDtypeStruct((nq, N_DOCS), jnp.int32),
        ],
    )(tv5, ti5, queries_sub, keys_tail)


def kernel(queries, keys):
    keys_tail = jnp.concatenate(
        [jax.lax.slice(keys, (MAIN, 0), (K, D)),
         jnp.zeros((BT - TAIL, D), jnp.float32)], axis=0)

    tv, ti, v3 = _run_sweep(_sweep2_body, NCAP, queries, keys, True)
    out_v, out_i, sus = pl.pallas_call(
        _merge2_body,
        out_shape=[
            jax.ShapeDtypeStruct((Q, N_DOCS), jnp.float32),
            jax.ShapeDtypeStruct((Q, N_DOCS), jnp.int32),
            jax.ShapeDtypeStruct((Q, 1), jnp.int32),
        ],
    )(tv, ti, v3, queries, keys_tail)

    sus = sus[:, 0]
    n_sus = jnp.sum(sus)

    def _repair(_):
        (rows,) = jnp.nonzero(sus, size=QR, fill_value=Q + 7)
        qs = jnp.take(queries, jnp.minimum(rows, Q - 1), axis=0)
        rep_v, rep_i = _depth5_topk(qs, keys, keys_tail)
        return (out_v.at[rows].set(rep_v, mode="drop"),
                out_i.at[rows].set(rep_i, mode="drop"))

    def _full(_):
        return _depth5_topk(queries, keys, keys_tail)

    def _suspect(_):
        return jax.lax.cond(n_sus <= QR, _repair, _full, None)

    return jax.lax.cond(
        n_sus == 0, lambda _: (out_v, out_i), _suspect, None)


# speedup vs baseline: 3.8199x; 1.0363x over previous
import jax
import jax.numpy as jnp
from jax.experimental import pallas as pl
from jax.experimental.pallas import tpu as pltpu

N_DOCS = 5
NCAP = 1
QR = 256
Q = 1024
D = 128
K = 100000
BK = 4096
NK_MAIN = K // BK
MAIN = NK_MAIN * BK
TAIL = K - MAIN
BT = 2048
CHUNK = 128
NCH = BK // CHUNK
NCH_TAIL = (TAIL + CHUNK - 1) // CHUNK

NEG_INF = float("-inf")
IMAX = jnp.iinfo(jnp.int32).max


def _dot(q, k):
    return jax.lax.dot_general(
        q, k, dimension_numbers=(((1,), (1,)), ((), ())),
        preferred_element_type=jnp.float32)


def _sweep2_body(q_ref, k_ref, tv_out, ti_out, v3_out, tv_ref, ti_ref, v3_ref):
    kb = pl.program_id(0)

    @pl.when(kb == 0)
    def _init():
        tv_ref[...] = jnp.full(tv_ref.shape, NEG_INF, jnp.float32)
        ti_ref[...] = jnp.zeros(ti_ref.shape, jnp.int32)
        v3_ref[...] = jnp.full(v3_ref.shape, NEG_INF, jnp.float32)

    s = _dot(q_ref[...], k_ref[...])

    for r in range(NCH):
        w = s[:, r * CHUNK:(r + 1) * CHUNK]
        wid = kb * NCH + r
        for t in range(NCAP):
            tv = tv_ref[t]
            ti = ti_ref[t]
            gt = w > tv
            tv_ref[t] = jnp.maximum(tv, w)
            ti_ref[t] = jnp.where(gt, wid, ti)
            if t < NCAP - 1:
                w, wid = jnp.minimum(tv, w), jnp.where(gt, ti, wid)
            else:
                w = jnp.minimum(tv, w)
        v3_ref[...] = jnp.maximum(v3_ref[...], w)

    @pl.when(kb == NK_MAIN - 1)
    def _flush():
        tv_out[...] = tv_ref[...]
        ti_out[...] = ti_ref[...]
        v3_out[...] = v3_ref[...]


def _sweep5_body(q_ref, k_ref, tv_out, ti_out, tv_ref, ti_ref):
    kb = pl.program_id(0)

    @pl.when(kb == 0)
    def _init():
        tv_ref[...] = jnp.full(tv_ref.shape, NEG_INF, jnp.float32)
        ti_ref[...] = jnp.zeros(ti_ref.shape, jnp.int32)

    s = _dot(q_ref[...], k_ref[...])

    for r in range(NCH):
        w = s[:, r * CHUNK:(r + 1) * CHUNK]
        wid = kb * NCH + r
        for t in range(N_DOCS):
            tv = tv_ref[t]
            ti = ti_ref[t]
            gt = w > tv
            tv_ref[t] = jnp.maximum(tv, w)
            ti_ref[t] = jnp.where(gt, wid, ti)
            if t < N_DOCS - 1:
                w, wid = jnp.minimum(tv, w), jnp.where(gt, ti, wid)

    @pl.when(kb == NK_MAIN - 1)
    def _flush():
        tv_out[...] = tv_ref[...]
        ti_out[...] = ti_ref[...]


def _tail_scores(q_ref, kt_ref):
    nq = q_ref.shape[0]
    s = _dot(q_ref[...], kt_ref[...])
    col = jax.lax.broadcasted_iota(jnp.int32, (nq, CHUNK), 1)
    out = []
    for r in range(NCH_TAIL):
        limit = TAIL - r * CHUNK
        w = s[:, r * CHUNK:(r + 1) * CHUNK]
        if limit < CHUNK:
            w = jnp.where(col < limit, w, NEG_INF)
        out.append((w, MAIN // CHUNK + r))
    return out


def _select_top5(tvs, tis, out_v_ref, out_i_ref):
    nq = tvs[0].shape[0]
    depth = len(tvs)
    cv = jnp.concatenate(tvs, axis=1)
    cc = jnp.concatenate(tis, axis=1)
    lane = jax.lax.rem(
        jax.lax.broadcasted_iota(jnp.int32, (nq, depth * CHUNK), 1), CHUNK)
    ci = cc * CHUNK + lane
    x5 = None
    for t in range(N_DOCS):
        m = jnp.max(cv, axis=1, keepdims=True)
        hit = cv == m
        sel = jnp.min(jnp.where(hit, ci, IMAX), axis=1, keepdims=True)
        out_v_ref[:, pl.ds(t, 1)] = m
        out_i_ref[:, pl.ds(t, 1)] = sel
        cv = jnp.where(hit & (ci == sel), NEG_INF, cv)
        x5 = m
    return x5


def _merge2_body(tv_ref, ti_ref, v3_ref, q_ref, kt_ref,
                 out_v_ref, out_i_ref, sus_ref):
    tvs = [tv_ref[t] for t in range(NCAP)]
    tis = [ti_ref[t] for t in range(NCAP)]
    v3 = v3_ref[...]
    for w, wid in _tail_scores(q_ref, kt_ref):
        for t in range(NCAP):
            gt = w > tvs[t]
            tvs[t], w = jnp.maximum(tvs[t], w), jnp.minimum(tvs[t], w)
            tis[t], wid = (jnp.where(gt, wid, tis[t]),
                           jnp.where(gt, tis[t], wid))
        v3 = jnp.maximum(v3, w)

    x5 = _select_top5(tvs, tis, out_v_ref, out_i_ref)
    mv3 = jnp.max(v3, axis=1, keepdims=True)
    sus_ref[...] = (mv3 >= x5).astype(jnp.int32)


def _merge5_body(tv_ref, ti_ref, q_ref, kt_ref, out_v_ref, out_i_ref):
    depth = tv_ref.shape[0]
    tvs = [tv_ref[t] for t in range(depth)]
    tis = [ti_ref[t] for t in range(depth)]
    for w, wid in _tail_scores(q_ref, kt_ref):
        for t in range(depth):
            gt = w > tvs[t]
            tvs[t], w = jnp.maximum(tvs[t], w), jnp.minimum(tvs[t], w)
            tis[t], wid = (jnp.where(gt, wid, tis[t]),
                           jnp.where(gt, tis[t], wid))
    _select_top5(tvs, tis, out_v_ref, out_i_ref)


def _run_sweep(body, depth, queries, keys, with_aux):
    nq = queries.shape[0]
    aux_out = ([pl.BlockSpec((nq, CHUNK), lambda k: (0, 0))]
               if with_aux else [])
    aux_shape = ([jax.ShapeDtypeStruct((nq, CHUNK), jnp.float32)]
                 if with_aux else [])
    aux_scratch = ([pltpu.VMEM((nq, CHUNK), jnp.float32)] if with_aux else [])
    return pl.pallas_call(
        body,
        grid=(NK_MAIN,),
        in_specs=[
            pl.BlockSpec((nq, D), lambda k: (0, 0)),
            pl.BlockSpec((BK, D), lambda k: (k, 0)),
        ],
        out_specs=[
            pl.BlockSpec((depth, nq, CHUNK), lambda k: (0, 0, 0)),
            pl.BlockSpec((depth, nq, CHUNK), lambda k: (0, 0, 0)),
        ] + aux_out,
        out_shape=[
            jax.ShapeDtypeStruct((depth, nq, CHUNK), jnp.float32),
            jax.ShapeDtypeStruct((depth, nq, CHUNK), jnp.int32),
        ] + aux_shape,
        scratch_shapes=[
            pltpu.VMEM((depth, nq, CHUNK), jnp.float32),
            pltpu.VMEM((depth, nq, CHUNK), jnp.int32),
        ] + aux_scratch,
        compiler_params=pltpu.CompilerParams(
            dimension_semantics=("arbitrary",),
        ),
    )(queries, keys)


def _depth5_topk(queries_sub, keys, keys_tail):
    tv5, ti5 = _run_sweep(_sweep5_body, N_DOCS, queries_sub, keys, False)
    nq = queries_sub.shape[0]
    return pl.pallas_call(
        _merge5_body,
        out_shape=[
            jax.ShapeDtypeStruct((nq, N_DOCS), jnp.float32),
            jax.ShapeDtypeStruct((nq, N_DOCS), jnp.int32),
        ],
    )(tv5, ti5, queries_sub, keys_tail)


def kernel(queries, keys):
    keys_tail = jnp.concatenate(
        [jax.lax.slice(keys, (MAIN, 0), (K, D)),
         jnp.zeros((BT - TAIL, D), jnp.float32)], axis=0)

    tv, ti, v3 = _run_sweep(_sweep2_body, NCAP, queries, keys, True)
    out_v, out_i, sus = pl.pallas_call(
        _merge2_body,
        out_shape=[
            jax.ShapeDtypeStruct((Q, N_DOCS), jnp.float32),
            jax.ShapeDtypeStruct((Q, N_DOCS), jnp.int32),
            jax.ShapeDtypeStruct((Q, 1), jnp.int32),
        ],
    )(tv, ti, v3, queries, keys_tail)

    sus = sus[:, 0]
    n_sus = jnp.sum(sus)

    def _repair(_):
        (rows,) = jnp.nonzero(sus, size=QR, fill_value=Q + 7)
        qs = jnp.take(queries, jnp.minimum(rows, Q - 1), axis=0)
        rep_v, rep_i = _depth5_topk(qs, keys, keys_tail)
        return (out_v.at[rows].set(rep_v, mode="drop"),
                out_i.at[rows].set(rep_i, mode="drop"))

    def _full(_):
        return _depth5_topk(queries, keys, keys_tail)

    def _suspect(_):
        return jax.lax.cond(n_sus <= QR, _repair, _full, None)

    return jax.lax.cond(
        n_sus == 0, lambda _: (out_v, out_i), _suspect, None)


# speedup vs baseline: 4.7342x; 1.2394x over previous
import jax
import jax.numpy as jnp
from jax.experimental import pallas as pl
from jax.experimental.pallas import tpu as pltpu

N_DOCS = 5
NCAP = 1
QR = 128
Q = 1024
D = 128
K = 100000
BK = 4096
NK_MAIN = K // BK
MAIN = NK_MAIN * BK
TAIL = K - MAIN
BT = 2048
CHUNK = 128
NCH = BK // CHUNK
NCH_TAIL = (TAIL + CHUNK - 1) // CHUNK

NEG_INF = float("-inf")
IMAX = jnp.iinfo(jnp.int32).max


def _dot(q, k):
    return jax.lax.dot_general(
        q, k, dimension_numbers=(((1,), (1,)), ((), ())),
        preferred_element_type=jnp.float32)


def _sweep2_body(q_ref, k_ref, tv_out, ti_out, v3_out, tv_ref, ti_ref, v3_ref):
    kb = pl.program_id(0)

    @pl.when(kb == 0)
    def _init():
        tv_ref[...] = jnp.full(tv_ref.shape, NEG_INF, jnp.float32)
        ti_ref[...] = jnp.zeros(ti_ref.shape, jnp.int32)
        v3_ref[...] = jnp.full(v3_ref.shape, NEG_INF, jnp.float32)

    s = _dot(q_ref[...], k_ref[...])

    for r in range(NCH):
        w = s[:, r * CHUNK:(r + 1) * CHUNK]
        wid = kb * NCH + r
        for t in range(NCAP):
            tv = tv_ref[t]
            ti = ti_ref[t]
            gt = w > tv
            tv_ref[t] = jnp.maximum(tv, w)
            ti_ref[t] = jnp.where(gt, wid, ti)
            if t < NCAP - 1:
                w, wid = jnp.minimum(tv, w), jnp.where(gt, ti, wid)
            else:
                w = jnp.minimum(tv, w)
        v3_ref[...] = jnp.maximum(v3_ref[...], w)

    @pl.when(kb == NK_MAIN - 1)
    def _flush():
        tv_out[...] = tv_ref[...]
        ti_out[...] = ti_ref[...]
        v3_out[...] = v3_ref[...]


def _sweep5_body(q_ref, k_ref, tv_out, ti_out, tv_ref, ti_ref):
    kb = pl.program_id(0)

    @pl.when(kb == 0)
    def _init():
        tv_ref[...] = jnp.full(tv_ref.shape, NEG_INF, jnp.float32)
        ti_ref[...] = jnp.zeros(ti_ref.shape, jnp.int32)

    s = _dot(q_ref[...], k_ref[...])

    for r in range(NCH):
        w = s[:, r * CHUNK:(r + 1) * CHUNK]
        wid = kb * NCH + r
        for t in range(N_DOCS):
            tv = tv_ref[t]
            ti = ti_ref[t]
            gt = w > tv
            tv_ref[t] = jnp.maximum(tv, w)
            ti_ref[t] = jnp.where(gt, wid, ti)
            if t < N_DOCS - 1:
                w, wid = jnp.minimum(tv, w), jnp.where(gt, ti, wid)

    @pl.when(kb == NK_MAIN - 1)
    def _flush():
        tv_out[...] = tv_ref[...]
        ti_out[...] = ti_ref[...]


def _tail_scores(q_ref, kt_ref):
    nq = q_ref.shape[0]
    s = _dot(q_ref[...], kt_ref[...])
    col = jax.lax.broadcasted_iota(jnp.int32, (nq, CHUNK), 1)
    out = []
    for r in range(NCH_TAIL):
        limit = TAIL - r * CHUNK
        w = s[:, r * CHUNK:(r + 1) * CHUNK]
        if limit < CHUNK:
            w = jnp.where(col < limit, w, NEG_INF)
        out.append((w, MAIN // CHUNK + r))
    return out


def _select_top5(tvs, tis, out_v_ref, out_i_ref):
    nq = tvs[0].shape[0]
    depth = len(tvs)
    cv = jnp.concatenate(tvs, axis=1)
    cc = jnp.concatenate(tis, axis=1)
    lane = jax.lax.rem(
        jax.lax.broadcasted_iota(jnp.int32, (nq, depth * CHUNK), 1), CHUNK)
    ci = cc * CHUNK + lane
    x5 = None
    for t in range(N_DOCS):
        m = jnp.max(cv, axis=1, keepdims=True)
        hit = cv == m
        sel = jnp.min(jnp.where(hit, ci, IMAX), axis=1, keepdims=True)
        out_v_ref[:, pl.ds(t, 1)] = m
        out_i_ref[:, pl.ds(t, 1)] = sel
        cv = jnp.where(hit & (ci == sel), NEG_INF, cv)
        x5 = m
    return x5


def _merge2_body(tv_ref, ti_ref, v3_ref, q_ref, kt_ref,
                 out_v_ref, out_i_ref, sus_ref):
    tvs = [tv_ref[t] for t in range(NCAP)]
    tis = [ti_ref[t] for t in range(NCAP)]
    v3 = v3_ref[...]
    for w, wid in _tail_scores(q_ref, kt_ref):
        for t in range(NCAP):
            gt = w > tvs[t]
            tvs[t], w = jnp.maximum(tvs[t], w), jnp.minimum(tvs[t], w)
            tis[t], wid = (jnp.where(gt, wid, tis[t]),
                           jnp.where(gt, tis[t], wid))
        v3 = jnp.maximum(v3, w)

    x5 = _select_top5(tvs, tis, out_v_ref, out_i_ref)
    mv3 = jnp.max(v3, axis=1, keepdims=True)
    sus_ref[...] = (mv3 >= x5).astype(jnp.int32)


def _merge5_body(tv_ref, ti_ref, q_ref, kt_ref, out_v_ref, out_i_ref):
    depth = tv_ref.shape[0]
    tvs = [tv_ref[t] for t in range(depth)]
    tis = [ti_ref[t] for t in range(depth)]
    for w, wid in _tail_scores(q_ref, kt_ref):
        for t in range(depth):
            gt = w > tvs[t]
            tvs[t], w = jnp.maximum(tvs[t], w), jnp.minimum(tvs[t], w)
            tis[t], wid = (jnp.where(gt, wid, tis[t]),
                           jnp.where(gt, tis[t], wid))
    _select_top5(tvs, tis, out_v_ref, out_i_ref)


def _run_sweep(body, depth, queries, keys, with_aux):
    nq = queries.shape[0]
    aux_out = ([pl.BlockSpec((nq, CHUNK), lambda k: (0, 0))]
               if with_aux else [])
    aux_shape = ([jax.ShapeDtypeStruct((nq, CHUNK), jnp.float32)]
                 if with_aux else [])
    aux_scratch = ([pltpu.VMEM((nq, CHUNK), jnp.float32)] if with_aux else [])
    return pl.pallas_call(
        body,
        grid=(NK_MAIN,),
        in_specs=[
            pl.BlockSpec((nq, D), lambda k: (0, 0)),
            pl.BlockSpec((BK, D), lambda k: (k, 0)),
        ],
        out_specs=[
            pl.BlockSpec((depth, nq, CHUNK), lambda k: (0, 0, 0)),
            pl.BlockSpec((depth, nq, CHUNK), lambda k: (0, 0, 0)),
        ] + aux_out,
        out_shape=[
            jax.ShapeDtypeStruct((depth, nq, CHUNK), jnp.float32),
            jax.ShapeDtypeStruct((depth, nq, CHUNK), jnp.int32),
        ] + aux_shape,
        scratch_shapes=[
            pltpu.VMEM((depth, nq, CHUNK), jnp.float32),
            pltpu.VMEM((depth, nq, CHUNK), jnp.int32),
        ] + aux_scratch,
        compiler_params=pltpu.CompilerParams(
            dimension_semantics=("arbitrary",),
        ),
    )(queries, keys)


def _depth5_topk(queries_sub, keys, keys_tail):
    tv5, ti5 = _run_sweep(_sweep5_body, N_DOCS, queries_sub, keys, False)
    nq = queries_sub.shape[0]
    return pl.pallas_call(
        _merge5_body,
        out_shape=[
            jax.ShapeDtypeStruct((nq, N_DOCS), jnp.float32),
            jax.ShapeDtypeStruct((nq, N_DOCS), jnp.int32),
        ],
    )(tv5, ti5, queries_sub, keys_tail)


def kernel(queries, keys):
    keys_tail = jnp.concatenate(
        [jax.lax.slice(keys, (MAIN, 0), (K, D)),
         jnp.zeros((BT - TAIL, D), jnp.float32)], axis=0)

    tv, ti, v3 = _run_sweep(_sweep2_body, NCAP, queries, keys, True)
    out_v, out_i, sus = pl.pallas_call(
        _merge2_body,
        out_shape=[
            jax.ShapeDtypeStruct((Q, N_DOCS), jnp.float32),
            jax.ShapeDtypeStruct((Q, N_DOCS), jnp.int32),
            jax.ShapeDtypeStruct((Q, 1), jnp.int32),
        ],
    )(tv, ti, v3, queries, keys_tail)

    sus = sus[:, 0]
    n_sus = jnp.sum(sus)

    def _repair(_):
        (rows,) = jnp.nonzero(sus, size=QR, fill_value=Q + 7)
        qs = jnp.take(queries, jnp.minimum(rows, Q - 1), axis=0)
        rep_v, rep_i = _depth5_topk(qs, keys, keys_tail)
        return (out_v.at[rows].set(rep_v, mode="drop"),
                out_i.at[rows].set(rep_i, mode="drop"))

    def _full(_):
        return _depth5_topk(queries, keys, keys_tail)

    def _suspect(_):
        return jax.lax.cond(n_sus <= QR, _repair, _full, None)

    return jax.lax.cond(
        n_sus == 0, lambda _: (out_v, out_i), _suspect, None)
